# baseline (speedup 1.0000x reference)
import functools

import jax
import jax.numpy as jnp
from jax import lax
from jax.experimental import pallas as pl
from jax.experimental.pallas import tpu as pltpu
from jax.experimental.pallas import tpu_sc as plsc

L = 2
R = 16
G = 2048
H = 8
D = 128
T = 1024

ROWS = L * R * G
NW = 32
OUT_ROWS = 2 * L * T
MPW = (2 * T) // NW


def _body(k_hbm, v_hbm, req_hbm, tgt_hbm, src_hbm, out_hbm,
          req_v, tgt_v, src_v, table_v, idx_v, sem, sem_v):
    wid = lax.axis_index("s") * 2 + lax.axis_index("c")

    pltpu.sync_copy(req_hbm, req_v)
    pltpu.sync_copy(tgt_hbm, tgt_v)
    pltpu.sync_copy(src_hbm, src_v)

    lane = lax.broadcasted_iota(jnp.int32, (16,), 0)
    for c in range(T // 16):
        rq = req_v[pl.ds(c * 16, 16)]
        tg = tgt_v[pl.ds(c * 16, 16)]
        sv = src_v[pl.ds(c * 16, 16)]
        kv = rq * G + tg
        enc = (lane + c * 16) * G + sv
        plsc.store_scatter(table_v, [kv], enc)

        def _fix(go, kv=kv, enc=enc):
            del go
            g = plsc.load_gather(table_v, [kv])
            m = enc > g
            plsc.store_scatter(table_v, [kv], enc, mask=m)
            return jnp.any(m)

        lax.while_loop(lambda go: go, _fix, jnp.bool_(True))

    layer_off = (wid // 16) * (R * G)
    m0 = (wid % 16) * MPW
    for q in range(MPW // 16):
        ji = m0 + q * 16 + lane
        rq = plsc.load_gather(req_v, [ji])
        tg = plsc.load_gather(tgt_v, [ji])
        win = plsc.load_gather(table_v, [rq * G + tg])
        idx_v[pl.ds(q * 16, 16)] = layer_off + rq * G + (win & (G - 1))

    for c in range(MPW // 16):
        chunk = idx_v[pl.ds(c * 16, 16)]
        for j in range(16):
            row = chunk[j]
            i = c * 16 + j
            pltpu.async_copy(k_hbm.at[row],
                             out_hbm.at[wid * MPW + i], sem)
            pltpu.async_copy(v_hbm.at[row],
                             out_hbm.at[2 * T + wid * MPW + i], sem_v)

    pltpu.make_async_copy(
        k_hbm.at[pl.ds(0, MPW)],
        out_hbm.at[pl.ds(wid * MPW, MPW)], sem).wait()
    pltpu.make_async_copy(
        v_hbm.at[pl.ds(0, MPW)],
        out_hbm.at[pl.ds(2 * T + wid * MPW, MPW)], sem_v).wait()


def _run(k3, v3, req, tgt, src):
    mesh = plsc.VectorSubcoreMesh(core_axis_name="c", subcore_axis_name="s")
    fn = functools.partial(
        pl.kernel,
        mesh=mesh,
        compiler_params=pltpu.CompilerParams(needs_layout_passes=False),
        out_type=jax.ShapeDtypeStruct((OUT_ROWS, H, D), jnp.float16),
        scratch_types=[
            pltpu.VMEM((T,), jnp.int32),
            pltpu.VMEM((T,), jnp.int32),
            pltpu.VMEM((T,), jnp.int32),
            pltpu.VMEM((R * G,), jnp.int32),
            pltpu.VMEM((MPW,), jnp.int32),
            pltpu.SemaphoreType.DMA,
            pltpu.SemaphoreType.DMA,
        ],
    )(_body)
    return fn(k3, v3, req, tgt, src)


def kernel(K_cache, V_cache, req_indices, src_positions, tgt_positions):
    req = req_indices.astype(jnp.int32)
    tgt = tgt_positions.astype(jnp.int32)
    src = src_positions.astype(jnp.int32)
    k3 = K_cache.reshape(ROWS, H, D)
    v3 = V_cache.reshape(ROWS, H, D)
    out = _run(k3, v3, req, tgt, src)
    return out.reshape(2 * L, T, H, D)


# speedup vs baseline: 8.4506x; 8.4506x over previous
import functools

import jax
import jax.numpy as jnp
from jax import lax
from jax.experimental import pallas as pl
from jax.experimental.pallas import tpu as pltpu
from jax.experimental.pallas import tpu_sc as plsc

L = 2
R = 16
G = 2048
H = 8
D = 128
T = 1024

ROWS = L * R * G
NW = 32
OUT_ROWS = 2 * L * T
MPW = (2 * T) // NW


def _body(k_hbm, v_hbm, req_hbm, tgt_hbm, src_hbm, out_hbm,
          req_v, tgt_v, src_v, table_v, idx_v, rows_v,
          sk0, sk1, sk2, sk3, sv0, sv1, sv2, sv3, sem_out):
    wid = lax.axis_index("s") * 2 + lax.axis_index("c")

    pltpu.sync_copy(req_hbm, req_v)
    pltpu.sync_copy(tgt_hbm, tgt_v)
    pltpu.sync_copy(src_hbm, src_v)

    lane = lax.broadcasted_iota(jnp.int32, (16,), 0)
    for c in range(T // 16):
        rq = req_v[pl.ds(c * 16, 16)]
        tg = tgt_v[pl.ds(c * 16, 16)]
        sv = src_v[pl.ds(c * 16, 16)]
        kv = rq * G + tg
        enc = (lane + c * 16) * G + sv
        plsc.store_scatter(table_v, [kv], enc)

        def _fix(go, kv=kv, enc=enc):
            del go
            g = plsc.load_gather(table_v, [kv])
            m = enc > g
            plsc.store_scatter(table_v, [kv], enc, mask=m)
            return jnp.any(m)

        lax.while_loop(lambda go: go, _fix, jnp.bool_(True))

    layer_off = (wid // 16) * (R * G)
    m0 = (wid % 16) * MPW
    for q in range(MPW // 16):
        ji = m0 + q * 16 + lane
        rq = plsc.load_gather(req_v, [ji])
        tg = plsc.load_gather(tgt_v, [ji])
        win = plsc.load_gather(table_v, [rq * G + tg])
        idx_v[pl.ds(q * 16, 16)] = layer_off + rq * G + (win & (G - 1))

    ksems = [sk0, sk1, sk2, sk3]
    vsems = [sv0, sv1, sv2, sv3]
    for c in range(MPW // 16):
        chunk = idx_v[pl.ds(c * 16, 16)]
        for j in range(16):
            row = chunk[j]
            i = c * 16 + j
            pltpu.async_copy(k_hbm.at[row], rows_v.at[i], ksems[c])
            pltpu.async_copy(v_hbm.at[row], rows_v.at[MPW + i], vsems[c])

    outs = []
    for c in range(MPW // 16):
        pltpu.make_async_copy(
            k_hbm.at[pl.ds(0, 16)],
            rows_v.at[pl.ds(c * 16, 16)], ksems[c]).wait()
        outs.append(pltpu.async_copy(
            rows_v.at[pl.ds(c * 16, 16)],
            out_hbm.at[pl.ds(wid * MPW + c * 16, 16)], sem_out))
    for c in range(MPW // 16):
        pltpu.make_async_copy(
            v_hbm.at[pl.ds(0, 16)],
            rows_v.at[pl.ds(MPW + c * 16, 16)], vsems[c]).wait()
        outs.append(pltpu.async_copy(
            rows_v.at[pl.ds(MPW + c * 16, 16)],
            out_hbm.at[pl.ds(2 * T + wid * MPW + c * 16, 16)], sem_out))
    for o in outs:
        o.wait()


def _run(k3, v3, req, tgt, src):
    mesh = plsc.VectorSubcoreMesh(core_axis_name="c", subcore_axis_name="s")
    fn = functools.partial(
        pl.kernel,
        mesh=mesh,
        compiler_params=pltpu.CompilerParams(needs_layout_passes=False),
        out_type=jax.ShapeDtypeStruct((OUT_ROWS, H, D), jnp.float16),
        scratch_types=[
            pltpu.VMEM((T,), jnp.int32),
            pltpu.VMEM((T,), jnp.int32),
            pltpu.VMEM((T,), jnp.int32),
            pltpu.VMEM((R * G,), jnp.int32),
            pltpu.VMEM((MPW,), jnp.int32),
            pltpu.VMEM((2 * MPW, H, D), jnp.float16),
        ] + [pltpu.SemaphoreType.DMA] * 9,
    )(_body)
    return fn(k3, v3, req, tgt, src)


def kernel(K_cache, V_cache, req_indices, src_positions, tgt_positions):
    req = req_indices.astype(jnp.int32)
    tgt = tgt_positions.astype(jnp.int32)
    src = src_positions.astype(jnp.int32)
    k3 = K_cache.reshape(ROWS, H, D)
    v3 = V_cache.reshape(ROWS, H, D)
    out = _run(k3, v3, req, tgt, src)
    return out.reshape(2 * L, T, H, D)
